# baseline (speedup 1.0000x reference)
import functools

import jax
import jax.numpy as jnp
from jax import lax
from jax.experimental import pallas as pl
from jax.experimental.pallas import tpu as pltpu
from jax.experimental.pallas import tpu_sc as plsc

NUM_ROWS = 1_000_000
DIM = 32
NUM_CORES = 2
NUM_SUBCORES = 16
NUM_WORKERS = NUM_CORES * NUM_SUBCORES

LANE = 128
TILES = NUM_ROWS // LANE
TILES_PER_WORKER = TILES // NUM_WORKERS
EXTRA_TILES = TILES - TILES_PER_WORKER * NUM_WORKERS
TAIL_WORKER = EXTRA_TILES

CHUNKS_A = [8] * 15 + [5]
CHUNKS_B = [7] * 17
OFF_B = sum(CHUNKS_A)
MAXC_A = 8 * LANE
MAXC_B = 7 * LANE
NBUF = 2

_MESH = plsc.VectorSubcoreMesh(core_axis_name="c", subcore_axis_name="s")


@functools.partial(
    pl.kernel,
    mesh=_MESH,
    out_type=jax.ShapeDtypeStruct((DIM, NUM_ROWS), jnp.float32),
    scratch_types=[
        pltpu.VMEM((DIM, MAXC_A), jnp.float32),
        pltpu.VMEM((DIM, MAXC_A), jnp.float32),
        pltpu.VMEM_SHARED((NUM_SUBCORES, DIM, MAXC_B), jnp.float32),
        pltpu.VMEM_SHARED((NUM_SUBCORES, DIM, MAXC_B), jnp.float32),
        pltpu.VMEM((DIM, LANE), jnp.float32),
        pltpu.SemaphoreType.DMA((NBUF,)),
        pltpu.SemaphoreType.DMA((NBUF,)),
        pltpu.SemaphoreType.DMA((NBUF,)),
        pltpu.SemaphoreType.DMA((NBUF,)),
        pltpu.SemaphoreType.DMA,
    ],
)
def _copy_kernel(
    in_hbm, out_hbm, bufa0, bufa1, bufb0, bufb1, tail_buf,
    in_sems_a, out_sems_a, in_sems_b, out_sems_b, tail_sem,
):
    wid = lax.axis_index("s") * NUM_CORES + lax.axis_index("c")
    sid = lax.axis_index("s")
    base = wid * (TILES_PER_WORKER * LANE)

    class Stream:
        def __init__(self, chunks, tile_off, bufs, in_sems, out_sems, maxc):
            self.chunks = chunks
            self.offs = [tile_off + sum(chunks[:k]) for k in range(len(chunks))]
            self.n = len(chunks)
            self.bufs = bufs
            self.in_sems = in_sems
            self.out_sems = out_sems
            self.maxc = maxc

        def hbm_slice(self, ref, k):
            start = pl.multiple_of(base + self.offs[k] * LANE, LANE)
            return ref.at[:, pl.ds(start, self.chunks[k] * LANE)]

        def buf(self, k):
            b = self.bufs[k % NBUF]
            cols = self.chunks[k] * LANE
            return b if cols == self.maxc else b.at[:, :cols]

        def copy_in(self, k):
            return pltpu.make_async_copy(
                self.hbm_slice(in_hbm, k), self.buf(k), self.in_sems.at[k % NBUF]
            )

        def copy_out(self, k):
            return pltpu.make_async_copy(
                self.buf(k), self.hbm_slice(out_hbm, k), self.out_sems.at[k % NBUF]
            )

    streams = (
        Stream(CHUNKS_A, 0, (bufa0, bufa1), in_sems_a, out_sems_a, MAXC_A),
        Stream(CHUNKS_B, OFF_B, (bufb0.at[sid], bufb1.at[sid]), in_sems_b, out_sems_b, MAXC_B),
    )

    def tail_slice(ref):
        extra = (TILES_PER_WORKER * NUM_WORKERS + wid) * LANE
        start = pl.multiple_of(jnp.where(wid == TAIL_WORKER, TILES * LANE, extra), LANE)
        return ref.at[:, pl.ds(start, LANE)]

    has_tail = wid <= TAIL_WORKER

    pl.when(has_tail)(
        lambda: pltpu.make_async_copy(tail_slice(in_hbm), tail_buf, tail_sem).start()
    )

    for j in range(NBUF):
        for s in streams:
            if j < s.n:
                s.copy_in(j).start()
    for k in range(max(s.n for s in streams)):
        for s in streams:
            if k < s.n:
                s.copy_in(k).wait()
                s.copy_out(k).start()
                if k + NBUF < s.n:
                    s.copy_out(k).wait()
                    s.copy_in(k + NBUF).start()
    for s in streams:
        for k in range(max(0, s.n - NBUF), s.n):
            s.copy_out(k).wait()

    @pl.when(has_tail)
    def _tail():
        pltpu.make_async_copy(tail_slice(in_hbm), tail_buf, tail_sem).wait()
        pltpu.make_async_copy(tail_buf, tail_slice(out_hbm), tail_sem).start()
        pltpu.make_async_copy(tail_buf, tail_slice(out_hbm), tail_sem).wait()


def kernel(embedding):
    return _copy_kernel(embedding.T).T


# speedup vs baseline: 1.0307x; 1.0307x over previous
import functools

import jax
import jax.numpy as jnp
from jax import lax
from jax.experimental import pallas as pl
from jax.experimental.pallas import tpu as pltpu
from jax.experimental.pallas import tpu_sc as plsc

NUM_ROWS = 1_000_000
DIM = 32
NUM_CORES = 2
NUM_SUBCORES = 16
NUM_WORKERS = NUM_CORES * NUM_SUBCORES

LANE = 128
TILES = NUM_ROWS // LANE
TILES_PER_WORKER = TILES // NUM_WORKERS
EXTRA_TILES = TILES - TILES_PER_WORKER * NUM_WORKERS
TAIL_WORKER = EXTRA_TILES

CHUNK_TILES = [10] * 24 + [4]
CHUNK_OFFS = [sum(CHUNK_TILES[:k]) for k in range(len(CHUNK_TILES))]
NCHUNKS = len(CHUNK_TILES)
MAXC = max(CHUNK_TILES) * LANE
NBUF = 3

_MESH = plsc.VectorSubcoreMesh(core_axis_name="c", subcore_axis_name="s")


@functools.partial(
    pl.kernel,
    mesh=_MESH,
    out_type=jax.ShapeDtypeStruct((DIM, NUM_ROWS), jnp.float32),
    scratch_types=[
        pltpu.VMEM_SHARED((NUM_SUBCORES, DIM, MAXC), jnp.float32),
        pltpu.VMEM_SHARED((NUM_SUBCORES, DIM, MAXC), jnp.float32),
        pltpu.VMEM_SHARED((NUM_SUBCORES, DIM, MAXC), jnp.float32),
        pltpu.VMEM((DIM, LANE), jnp.float32),
        pltpu.SemaphoreType.DMA((NBUF,)),
        pltpu.SemaphoreType.DMA((NBUF,)),
        pltpu.SemaphoreType.DMA,
    ],
)
def _copy_kernel(in_hbm, out_hbm, buf0, buf1, buf2, tail_buf, in_sems, out_sems, tail_sem):
    wid = lax.axis_index("s") * NUM_CORES + lax.axis_index("c")
    sid = lax.axis_index("s")
    base = wid * (TILES_PER_WORKER * LANE)
    bufs = (buf0.at[sid], buf1.at[sid], buf2.at[sid])

    def hbm_slice(ref, k):
        start = pl.multiple_of(base + CHUNK_OFFS[k] * LANE, LANE)
        return ref.at[:, pl.ds(start, CHUNK_TILES[k] * LANE)]

    def vmem_buf(k):
        b = bufs[k % NBUF]
        cols = CHUNK_TILES[k] * LANE
        return b if cols == MAXC else b.at[:, :cols]

    def copy_in(k):
        return pltpu.make_async_copy(
            hbm_slice(in_hbm, k), vmem_buf(k), in_sems.at[k % NBUF]
        )

    def copy_out(k):
        return pltpu.make_async_copy(
            vmem_buf(k), hbm_slice(out_hbm, k), out_sems.at[k % NBUF]
        )

    def tail_slice(ref):
        extra = (TILES_PER_WORKER * NUM_WORKERS + wid) * LANE
        start = pl.multiple_of(jnp.where(wid == TAIL_WORKER, TILES * LANE, extra), LANE)
        return ref.at[:, pl.ds(start, LANE)]

    has_tail = wid <= TAIL_WORKER

    pl.when(has_tail)(
        lambda: pltpu.make_async_copy(tail_slice(in_hbm), tail_buf, tail_sem).start()
    )

    for j in range(NBUF):
        copy_in(j).start()
    for k in range(NCHUNKS):
        copy_in(k).wait()
        copy_out(k).start()
        if k + NBUF < NCHUNKS:
            copy_out(k).wait()
            copy_in(k + NBUF).start()
    for k in range(max(0, NCHUNKS - NBUF), NCHUNKS):
        copy_out(k).wait()

    @pl.when(has_tail)
    def _tail():
        pltpu.make_async_copy(tail_slice(in_hbm), tail_buf, tail_sem).wait()
        pltpu.make_async_copy(tail_buf, tail_slice(out_hbm), tail_sem).start()
        pltpu.make_async_copy(tail_buf, tail_slice(out_hbm), tail_sem).wait()


def kernel(embedding):
    return _copy_kernel(embedding.T).T


# speedup vs baseline: 1.0331x; 1.0024x over previous
import functools

import jax
import jax.numpy as jnp
from jax import lax
from jax.experimental import pallas as pl
from jax.experimental.pallas import tpu as pltpu
from jax.experimental.pallas import tpu_sc as plsc

NUM_ROWS = 1_000_000
DIM = 32
NUM_CORES = 2
NUM_SUBCORES = 16
NUM_WORKERS = NUM_CORES * NUM_SUBCORES

LANE = 128
TILES = NUM_ROWS // LANE
TILES_PER_WORKER = TILES // NUM_WORKERS
EXTRA_TILES = TILES - TILES_PER_WORKER * NUM_WORKERS
TAIL_WORKER = EXTRA_TILES

CHUNK_TILES = [15] * 16 + [4]
CHUNK_OFFS = [sum(CHUNK_TILES[:k]) for k in range(len(CHUNK_TILES))]
NCHUNKS = len(CHUNK_TILES)
MAXC = max(CHUNK_TILES) * LANE
NBUF = 2

_MESH = plsc.VectorSubcoreMesh(core_axis_name="c", subcore_axis_name="s")


@functools.partial(
    pl.kernel,
    mesh=_MESH,
    out_type=jax.ShapeDtypeStruct((DIM, NUM_ROWS), jnp.float32),
    scratch_types=[
        pltpu.VMEM_SHARED((NUM_SUBCORES, DIM, MAXC), jnp.float32),
        pltpu.VMEM_SHARED((NUM_SUBCORES, DIM, MAXC), jnp.float32),
        pltpu.VMEM((DIM, LANE), jnp.float32),
        pltpu.SemaphoreType.DMA((NBUF,)),
        pltpu.SemaphoreType.DMA((NBUF,)),
        pltpu.SemaphoreType.DMA,
    ],
)
def _copy_kernel(in_hbm, out_hbm, buf0, buf1, tail_buf, in_sems, out_sems, tail_sem):
    wid = lax.axis_index("s") * NUM_CORES + lax.axis_index("c")
    sid = lax.axis_index("s")
    base = wid * (TILES_PER_WORKER * LANE)
    bufs = (buf0.at[sid], buf1.at[sid])

    def hbm_slice(ref, k):
        start = pl.multiple_of(base + CHUNK_OFFS[k] * LANE, LANE)
        return ref.at[:, pl.ds(start, CHUNK_TILES[k] * LANE)]

    def vmem_buf(k):
        b = bufs[k % NBUF]
        cols = CHUNK_TILES[k] * LANE
        return b if cols == MAXC else b.at[:, :cols]

    def copy_in(k):
        return pltpu.make_async_copy(
            hbm_slice(in_hbm, k), vmem_buf(k), in_sems.at[k % NBUF]
        )

    def copy_out(k):
        return pltpu.make_async_copy(
            vmem_buf(k), hbm_slice(out_hbm, k), out_sems.at[k % NBUF]
        )

    def tail_slice(ref):
        extra = (TILES_PER_WORKER * NUM_WORKERS + wid) * LANE
        start = pl.multiple_of(jnp.where(wid == TAIL_WORKER, TILES * LANE, extra), LANE)
        return ref.at[:, pl.ds(start, LANE)]

    has_tail = wid <= TAIL_WORKER

    pl.when(has_tail)(
        lambda: pltpu.make_async_copy(tail_slice(in_hbm), tail_buf, tail_sem).start()
    )

    for j in range(NBUF):
        copy_in(j).start()
    for k in range(NCHUNKS):
        copy_in(k).wait()
        copy_out(k).start()
        if k + NBUF < NCHUNKS:
            copy_out(k).wait()
            copy_in(k + NBUF).start()
    for k in range(max(0, NCHUNKS - NBUF), NCHUNKS):
        copy_out(k).wait()

    @pl.when(has_tail)
    def _tail():
        pltpu.make_async_copy(tail_slice(in_hbm), tail_buf, tail_sem).wait()
        pltpu.make_async_copy(tail_buf, tail_slice(out_hbm), tail_sem).start()
        pltpu.make_async_copy(tail_buf, tail_slice(out_hbm), tail_sem).wait()


def kernel(embedding):
    return _copy_kernel(embedding.T).T
